# Initial kernel scaffold; baseline (speedup 1.0000x reference)
import jax
import jax.numpy as jnp
from jax.experimental import pallas as pl


def kernel(x, W_enc, b_enc, W_dec, b_dec):
    raise NotImplementedError("write your pallas kernel here")


# speedup vs baseline: 8.9583x; 8.9583x over previous
import functools

import jax
import jax.numpy as jnp
from jax.experimental import pallas as pl
from jax.experimental.pallas import tpu as pltpu

D_MODEL = 2048
D_SAE = 16384
N_TOK = 4096
TOPK = 64


def _encode_body(x_ref, w_ref, b_ref, out_ref):
    acc = jax.lax.dot_general(
        x_ref[...], w_ref[...],
        dimension_numbers=(((1,), (1,)), ((), ())),
        preferred_element_type=jnp.float32,
    )
    out_ref[...] = acc + b_ref[...]


def _encode(x, W_enc, b_enc, tb=512, sb=2048):
    grid = (D_SAE // sb, N_TOK // tb)
    return pl.pallas_call(
        _encode_body,
        grid=grid,
        in_specs=[
            pl.BlockSpec((tb, D_MODEL), lambda j, i: (i, 0)),
            pl.BlockSpec((sb, D_MODEL), lambda j, i: (j, 0)),
            pl.BlockSpec((1, sb), lambda j, i: (0, j)),
        ],
        out_specs=pl.BlockSpec((tb, sb), lambda j, i: (i, j)),
        out_shape=jax.ShapeDtypeStruct((N_TOK, D_SAE), jnp.float32),
        compiler_params=pltpu.CompilerParams(
            dimension_semantics=("arbitrary", "arbitrary"),
        ),
    )(x, W_enc, b_enc.reshape(1, D_SAE))


def _mono_u32(v):
    b = pltpu.bitcast(v, jnp.uint32)
    return jnp.where(b < jnp.uint32(0x80000000),
                     b ^ jnp.uint32(0x80000000),
                     ~b)


def _unmono_f32(u):
    b = jnp.where(u >= jnp.uint32(0x80000000), u ^ jnp.uint32(0x80000000), ~u)
    return pltpu.bitcast(b, jnp.float32)


def _thresh_body(lat_ref, thr_ref):
    mono = _mono_u32(lat_ref[...])
    tb = mono.shape[0]
    lo0 = jnp.zeros((tb, 1), dtype=jnp.uint32)

    def step(i, lo):
        bit = jnp.uint32(1) << (jnp.uint32(31) - jnp.uint32(i))
        mid = lo | bit
        cnt = jnp.sum((mono >= mid).astype(jnp.int32), axis=1, keepdims=True)
        return jnp.where(cnt >= TOPK, mid, lo)

    lo = jax.lax.fori_loop(0, 32, step, lo0)
    thr_ref[...] = _unmono_f32(lo)


def _thresholds(latents, tb=128):
    return pl.pallas_call(
        _thresh_body,
        grid=(N_TOK // tb,),
        in_specs=[pl.BlockSpec((tb, D_SAE), lambda i: (i, 0))],
        out_specs=pl.BlockSpec((tb, 1), lambda i: (i, 0)),
        out_shape=jax.ShapeDtypeStruct((N_TOK, 1), jnp.float32),
    )(latents)


def _decode_body(lat_ref, thr_ref, w_ref, b_ref, sparse_ref, recon_ref):
    k = pl.program_id(1)
    sparse = jnp.where(lat_ref[...] >= thr_ref[...], lat_ref[...], 0.0)
    sparse_ref[...] = sparse
    partial = jax.lax.dot_general(
        sparse, w_ref[...],
        dimension_numbers=(((1,), (1,)), ((), ())),
        preferred_element_type=jnp.float32,
    )

    @pl.when(k == 0)
    def _init():
        recon_ref[...] = partial + b_ref[...]

    @pl.when(k != 0)
    def _acc():
        recon_ref[...] += partial


def _decode(latents, thr, W_dec, b_dec, tb=256, kb=2048):
    grid = (N_TOK // tb, D_SAE // kb)
    return pl.pallas_call(
        _decode_body,
        grid=grid,
        in_specs=[
            pl.BlockSpec((tb, kb), lambda i, k: (i, k)),
            pl.BlockSpec((tb, 1), lambda i, k: (i, 0)),
            pl.BlockSpec((D_MODEL, kb), lambda i, k: (0, k)),
            pl.BlockSpec((1, D_MODEL), lambda i, k: (0, 0)),
        ],
        out_specs=[
            pl.BlockSpec((tb, kb), lambda i, k: (i, k)),
            pl.BlockSpec((tb, D_MODEL), lambda i, k: (i, 0)),
        ],
        out_shape=[
            jax.ShapeDtypeStruct((N_TOK, D_SAE), jnp.float32),
            jax.ShapeDtypeStruct((N_TOK, D_MODEL), jnp.float32),
        ],
        compiler_params=pltpu.CompilerParams(
            dimension_semantics=("arbitrary", "arbitrary"),
        ),
    )(latents, thr, W_dec, b_dec.reshape(1, D_MODEL))


@jax.jit
def kernel(x, W_enc, b_enc, W_dec, b_dec):
    latents = _encode(x, W_enc, b_enc)
    thr = _thresholds(latents)
    sparse_latents, recon = _decode(latents, thr, W_dec, b_dec)
    return recon, sparse_latents


# speedup vs baseline: 10.5673x; 1.1796x over previous
import functools

import jax
import jax.numpy as jnp
from jax.experimental import pallas as pl
from jax.experimental.pallas import tpu as pltpu

D_MODEL = 2048
D_SAE = 16384
N_TOK = 4096
TOPK = 64


def _encode_body(x_ref, w_ref, b_ref, out_ref):
    acc = jax.lax.dot_general(
        x_ref[...], w_ref[...],
        dimension_numbers=(((1,), (1,)), ((), ())),
        preferred_element_type=jnp.float32,
    )
    out_ref[...] = acc + b_ref[...]


def _encode(x, W_enc, b_enc, tb=512, sb=2048):
    grid = (D_SAE // sb, N_TOK // tb)
    return pl.pallas_call(
        _encode_body,
        grid=grid,
        in_specs=[
            pl.BlockSpec((tb, D_MODEL), lambda j, i: (i, 0)),
            pl.BlockSpec((sb, D_MODEL), lambda j, i: (j, 0)),
            pl.BlockSpec((1, sb), lambda j, i: (0, j)),
        ],
        out_specs=pl.BlockSpec((tb, sb), lambda j, i: (i, j)),
        out_shape=jax.ShapeDtypeStruct((N_TOK, D_SAE), jnp.float32),
        compiler_params=pltpu.CompilerParams(
            dimension_semantics=("arbitrary", "arbitrary"),
        ),
    )(x, W_enc, b_enc.reshape(1, D_SAE))


def _mono_u32(v):
    b = pltpu.bitcast(v, jnp.uint32)
    return jnp.where(b < jnp.uint32(0x80000000),
                     b ^ jnp.uint32(0x80000000),
                     ~b)


def _unmono_f32(u):
    b = jnp.where(u >= jnp.uint32(0x80000000), u ^ jnp.uint32(0x80000000), ~u)
    return pltpu.bitcast(b, jnp.float32)


def _thresh_body(lat_ref, thr_ref):
    mono = _mono_u32(lat_ref[...])
    tb = mono.shape[0]
    lo0 = jnp.zeros((tb, 1), dtype=jnp.uint32)

    def step(i, lo):
        bit = jnp.uint32(1) << (jnp.uint32(31) - jnp.uint32(i))
        mid = lo | bit
        cnt = jnp.sum((mono >= mid).astype(jnp.int32), axis=1, keepdims=True)
        return jnp.where(cnt >= TOPK, mid, lo)

    lo = jax.lax.fori_loop(0, 32, step, lo0)
    thr_ref[...] = _unmono_f32(lo)


def _thresholds(latents, tb=128):
    return pl.pallas_call(
        _thresh_body,
        grid=(N_TOK // tb,),
        in_specs=[pl.BlockSpec((tb, D_SAE), lambda i: (i, 0))],
        out_specs=pl.BlockSpec((tb, 1), lambda i: (i, 0)),
        out_shape=jax.ShapeDtypeStruct((N_TOK, 1), jnp.float32),
    )(latents)


def _decode_body(lat_ref, thr_ref, w_ref, b_ref, sparse_ref, recon_ref):
    k = pl.program_id(1)
    sparse = jnp.where(lat_ref[...] >= thr_ref[...], lat_ref[...], 0.0)
    sparse_ref[...] = sparse
    partial = jax.lax.dot_general(
        sparse.astype(jnp.bfloat16), w_ref[...],
        dimension_numbers=(((1,), (1,)), ((), ())),
        preferred_element_type=jnp.float32,
    )

    @pl.when(k == 0)
    def _init():
        recon_ref[...] = partial + b_ref[...]

    @pl.when(k != 0)
    def _acc():
        recon_ref[...] += partial


def _decode(latents, thr, W_dec, b_dec, tb=512, kb=2048):
    grid = (N_TOK // tb, D_SAE // kb)
    return pl.pallas_call(
        _decode_body,
        grid=grid,
        in_specs=[
            pl.BlockSpec((tb, kb), lambda i, k: (i, k)),
            pl.BlockSpec((tb, 1), lambda i, k: (i, 0)),
            pl.BlockSpec((D_MODEL, kb), lambda i, k: (0, k)),
            pl.BlockSpec((1, D_MODEL), lambda i, k: (0, 0)),
        ],
        out_specs=[
            pl.BlockSpec((tb, kb), lambda i, k: (i, k)),
            pl.BlockSpec((tb, D_MODEL), lambda i, k: (i, 0)),
        ],
        out_shape=[
            jax.ShapeDtypeStruct((N_TOK, D_SAE), jnp.float32),
            jax.ShapeDtypeStruct((N_TOK, D_MODEL), jnp.float32),
        ],
        compiler_params=pltpu.CompilerParams(
            dimension_semantics=("arbitrary", "arbitrary"),
        ),
    )(latents, thr, W_dec.astype(jnp.bfloat16), b_dec.reshape(1, D_MODEL))


@jax.jit
def kernel(x, W_enc, b_enc, W_dec, b_dec):
    latents = _encode(x, W_enc, b_enc)
    thr = _thresholds(latents)
    sparse_latents, recon = _decode(latents, thr, W_dec, b_dec)
    return recon, sparse_latents
